# baseline (speedup 1.0000x reference)
import functools

import jax
import jax.numpy as jnp
from jax import lax
from jax.experimental import pallas as pl
from jax.experimental.pallas import tpu as pltpu
from jax.experimental.pallas import tpu_sc as plsc

ROWS, COLS = 16384, 256
L = 16
GROUPS = COLS // L
CHUNK = 128


def kernel(x, permutation):
    info = plsc.get_sparse_core_info()
    nc, ns = info.num_cores, info.num_subcores
    nw = nc * ns
    rows_per_w = ROWS // nw
    nchunk = rows_per_w // CHUNK

    mesh = plsc.VectorSubcoreMesh(core_axis_name="c", subcore_axis_name="s")

    @functools.partial(
        pl.kernel,
        mesh=mesh,
        out_type=jax.ShapeDtypeStruct((ROWS, COLS), jnp.float32),
        scratch_types=[
            pltpu.VMEM((COLS,), jnp.int32),
            pltpu.VMEM((CHUNK, COLS), jnp.float32),
            pltpu.VMEM((CHUNK, COLS), jnp.float32),
        ],
        compiler_params=pltpu.CompilerParams(needs_layout_passes=False),
    )
    def run(x_hbm, perm_hbm, out_hbm, perm_v, in_v, out_v):
        wid = lax.axis_index("s") * nc + lax.axis_index("c")
        base = wid * rows_per_w
        pltpu.sync_copy(perm_hbm, perm_v)
        perm_vecs = [perm_v[pl.ds(g * L, L)] for g in range(GROUPS)]

        def step(c, carry):
            row0 = base + c * CHUNK
            pltpu.sync_copy(x_hbm.at[pl.ds(row0, CHUNK)], in_v)

            @plsc.parallel_loop(0, CHUNK, 1, unroll=1)
            def body(r):
                row_idx = jnp.full((L,), r, dtype=jnp.int32)
                for g in range(GROUPS):
                    out_v[r, pl.ds(g * L, L)] = plsc.load_gather(
                        in_v, [row_idx, perm_vecs[g]])

            pltpu.sync_copy(out_v, out_hbm.at[pl.ds(row0, CHUNK)])
            return carry

        lax.fori_loop(0, nchunk, step, 0)

    out = run(x, permutation)
    return out, jnp.zeros(ROWS, dtype=x.dtype)


# speedup vs baseline: 1.1828x; 1.1828x over previous
import functools

import jax
import jax.numpy as jnp
from jax import lax
from jax.experimental import pallas as pl
from jax.experimental.pallas import tpu as pltpu
from jax.experimental.pallas import tpu_sc as plsc

ROWS, COLS = 16384, 256
L = 16
GROUPS = COLS // L
CHUNK = 64


def kernel(x, permutation):
    info = plsc.get_sparse_core_info()
    nc, ns = info.num_cores, info.num_subcores
    nw = nc * ns
    rows_per_w = ROWS // nw
    nchunk = rows_per_w // CHUNK
    npair = nchunk // 2

    mesh = plsc.VectorSubcoreMesh(core_axis_name="c", subcore_axis_name="s")

    @functools.partial(
        pl.kernel,
        mesh=mesh,
        out_type=(
            jax.ShapeDtypeStruct((ROWS, COLS), jnp.float32),
            jax.ShapeDtypeStruct((ROWS,), jnp.float32),
        ),
        scratch_types=[
            pltpu.VMEM((COLS,), jnp.int32),
            pltpu.VMEM((rows_per_w,), jnp.float32),
            pltpu.VMEM((CHUNK, COLS), jnp.float32),
            pltpu.VMEM((CHUNK, COLS), jnp.float32),
            pltpu.VMEM((CHUNK, COLS), jnp.float32),
            pltpu.VMEM((CHUNK, COLS), jnp.float32),
            pltpu.SemaphoreType.DMA,
            pltpu.SemaphoreType.DMA,
            pltpu.SemaphoreType.DMA,
            pltpu.SemaphoreType.DMA,
        ],
        compiler_params=pltpu.CompilerParams(needs_layout_passes=False),
    )
    def run(x_hbm, perm_hbm, out_hbm, log_hbm, perm_v, zero_v,
            in0, in1, out0, out1, si0, si1, so0, so1):
        wid = lax.axis_index("s") * nc + lax.axis_index("c")
        base = wid * rows_per_w

        def in_copy(c, buf, sem):
            return pltpu.make_async_copy(
                x_hbm.at[pl.ds(base + c * CHUNK, CHUNK)], buf, sem)

        def out_copy(c, buf, sem):
            return pltpu.make_async_copy(
                buf, out_hbm.at[pl.ds(base + c * CHUNK, CHUNK)], sem)

        in_copy(0, in0, si0).start()
        in_copy(1, in1, si1).start()

        pltpu.sync_copy(perm_hbm, perm_v)
        perm_vecs = [perm_v[pl.ds(g * L, L)] for g in range(GROUPS)]

        zeros = jnp.zeros((L,), jnp.float32)

        @plsc.parallel_loop(0, rows_per_w // L, 1, unroll=1)
        def zbody(i):
            zero_v[pl.ds(i * L, L)] = zeros

        pltpu.sync_copy(zero_v, log_hbm.at[pl.ds(base, rows_per_w)])

        def compute(in_v, out_v):
            @plsc.parallel_loop(0, CHUNK, 1, unroll=1)
            def body(r):
                row_idx = jnp.full((L,), r, dtype=jnp.int32)
                for g in range(GROUPS):
                    out_v[r, pl.ds(g * L, L)] = plsc.load_gather(
                        in_v, [row_idx, perm_vecs[g]])

        def pair(k, carry):
            c0 = 2 * k

            in_copy(c0, in0, si0).wait()

            @pl.when(k >= 1)
            def _():
                out_copy(c0 - 2, out0, so0).wait()

            compute(in0, out0)
            out_copy(c0, out0, so0).start()

            @pl.when(k + 1 < npair)
            def _():
                in_copy(c0 + 2, in0, si0).start()

            in_copy(c0 + 1, in1, si1).wait()

            @pl.when(k >= 1)
            def _():
                out_copy(c0 - 1, out1, so1).wait()

            compute(in1, out1)
            out_copy(c0 + 1, out1, so1).start()

            @pl.when(k + 1 < npair)
            def _():
                in_copy(c0 + 3, in1, si1).start()

            return carry

        lax.fori_loop(0, npair, pair, 0)

        out_copy(nchunk - 2, out0, so0).wait()
        out_copy(nchunk - 1, out1, so1).wait()

    out, log_det = run(x, permutation)
    return out, log_det
